# baseline (speedup 1.0000x reference)
import jax
import jax.numpy as jnp
from jax import lax
from jax.experimental import pallas as pl
from jax.experimental.pallas import tpu as pltpu
from jax.experimental.pallas import tpu_sc as plsc

_M = 1000000
_K = 32
_B = 16384
_NC = 2
_NS = 16
_NW = _NC * _NS
_BPW = _B // _NW
_L = 16
_C = 128
_NCH = _BPW // _C


def _body(users_hbm, items_hbm, neg_hbm, ptf_hbm, qtf_hbm, pos_out, neg_out,
          idx_u, idx_i, idx_n, fu, fi, fn, du, di, dn, pos_v, neg_v, sem):
    wid = lax.axis_index("s") * _NC + lax.axis_index("c")
    base = wid * _BPW

    pltpu.sync_copy(users_hbm.at[pl.ds(base, _BPW)], idx_u)
    pltpu.sync_copy(items_hbm.at[pl.ds(base, _BPW)], idx_i)
    pltpu.sync_copy(neg_hbm.at[pl.ds(base, _BPW)], idx_n)

    def chunk(c, carry):
        def seg(s, carry2):
            sl = pl.ds(s * _L, _L)
            vu = idx_u[pl.ds(c * _C + s * _L, _L)]
            vi = idx_i[pl.ds(c * _C + s * _L, _L)]
            vn = idx_n[pl.ds(c * _C + s * _L, _L)]
            for k in range(_K):
                off = jnp.full((_L,), k * _M, jnp.int32)
                fu[k, sl] = vu + off
                fi[k, sl] = vi + off
                fn[k, sl] = vn + off
            return carry2
        lax.fori_loop(0, _C // _L, seg, 0)

        d = pl.ds(c * _C, _C)
        copies = []
        for k in range(_K):
            copies.append(pltpu.async_copy(
                ptf_hbm.at[fu.at[k]], du.at[k, d], sem))
            copies.append(pltpu.async_copy(
                qtf_hbm.at[fi.at[k]], di.at[k, d], sem))
            copies.append(pltpu.async_copy(
                qtf_hbm.at[fn.at[k]], dn.at[k, d], sem))
        for cp in copies:
            cp.wait()
        return carry
    lax.fori_loop(0, _NCH, chunk, 0)

    def group(g, carry):
        s = pl.ds(g * _L, _L)
        acc_p = jnp.zeros((_L,), jnp.float32)
        acc_n = jnp.zeros((_L,), jnp.float32)
        for k in range(_K):
            u = du[k, s]
            qi = di[k, s]
            qn = dn[k, s]
            acc_p = acc_p + u * qi
            acc_n = acc_n + u * qn
        pos_v[s] = acc_p
        neg_v[s] = acc_n
        return carry
    lax.fori_loop(0, _BPW // _L, group, 0)

    pltpu.sync_copy(pos_v, pos_out.at[pl.ds(base, _BPW)])
    pltpu.sync_copy(neg_v, neg_out.at[pl.ds(base, _BPW)])


@jax.jit
def _run(users, items, neg_items, p, q):
    ptf = p.T.reshape(_K * _M)
    qtf = q.T.reshape(_K * _M)
    mesh = plsc.VectorSubcoreMesh(core_axis_name="c", subcore_axis_name="s")
    f = pl.kernel(
        _body,
        mesh=mesh,
        out_type=(
            jax.ShapeDtypeStruct((_B,), jnp.float32),
            jax.ShapeDtypeStruct((_B,), jnp.float32),
        ),
        scratch_types=[
            pltpu.VMEM((_BPW,), jnp.int32),
            pltpu.VMEM((_BPW,), jnp.int32),
            pltpu.VMEM((_BPW,), jnp.int32),
            pltpu.VMEM((_K, _C), jnp.int32),
            pltpu.VMEM((_K, _C), jnp.int32),
            pltpu.VMEM((_K, _C), jnp.int32),
            pltpu.VMEM((_K, _BPW), jnp.float32),
            pltpu.VMEM((_K, _BPW), jnp.float32),
            pltpu.VMEM((_K, _BPW), jnp.float32),
            pltpu.VMEM((_BPW,), jnp.float32),
            pltpu.VMEM((_BPW,), jnp.float32),
            pltpu.SemaphoreType.DMA,
        ],
        compiler_params=pltpu.CompilerParams(
            needs_layout_passes=False, use_tc_tiling_on_sc=False
        ),
    )
    return f(users, items, neg_items, ptf, qtf)


def kernel(users, items, neg_items, P, Q):
    users = users.astype(jnp.int32)
    items = items.astype(jnp.int32)
    neg_items = neg_items.astype(jnp.int32)
    return _run(users, items, neg_items, P, Q)


# speedup vs baseline: 5.5897x; 5.5897x over previous
import jax
import jax.numpy as jnp
from jax import lax
from jax.experimental import pallas as pl
from jax.experimental.pallas import tpu as pltpu
from jax.experimental.pallas import tpu_sc as plsc

_M = 1000000
_K = 32
_B = 16384
_NC = 2
_NS = 16
_NW = _NC * _NS
_BPW = _B // _NW
_L = 16
_C = 128
_NCH = _BPW // _C


def _body(users_hbm, items_hbm, neg_hbm, p_hbm, q_hbm, pos_out, neg_out,
          idx_u, idx_i, idx_n, du, di, dn, pos_v, neg_v, sem):
    wid = lax.axis_index("s") * _NC + lax.axis_index("c")
    base = wid * _BPW

    pltpu.sync_copy(users_hbm.at[pl.ds(base, _BPW)], idx_u)
    pltpu.sync_copy(items_hbm.at[pl.ds(base, _BPW)], idx_i)
    pltpu.sync_copy(neg_hbm.at[pl.ds(base, _BPW)], idx_n)

    def chunk(c, carry):
        d = pl.ds(c * _C, _C)
        cp_u = pltpu.async_copy(p_hbm.at[idx_u.at[d]], du, sem)
        cp_i = pltpu.async_copy(q_hbm.at[idx_i.at[d]], di, sem)
        cp_n = pltpu.async_copy(q_hbm.at[idx_n.at[d]], dn, sem)
        cp_u.wait()
        cp_i.wait()
        cp_n.wait()

        def group(g, carry2):
            rows = g * _L + jnp.arange(_L, dtype=jnp.int32)
            acc_p = jnp.zeros((_L,), jnp.float32)
            acc_n = jnp.zeros((_L,), jnp.float32)
            for k in range(_K):
                col = jnp.full((_L,), k, jnp.int32)
                u = plsc.load_gather(du, [rows, col])
                acc_p = acc_p + u * plsc.load_gather(di, [rows, col])
                acc_n = acc_n + u * plsc.load_gather(dn, [rows, col])
            o = pl.ds(c * _C + g * _L, _L)
            pos_v[o] = acc_p
            neg_v[o] = acc_n
            return carry2
        lax.fori_loop(0, _C // _L, group, 0)
        return carry
    lax.fori_loop(0, _NCH, chunk, 0)

    pltpu.sync_copy(pos_v, pos_out.at[pl.ds(base, _BPW)])
    pltpu.sync_copy(neg_v, neg_out.at[pl.ds(base, _BPW)])


@jax.jit
def _run(users, items, neg_items, p, q):
    mesh = plsc.VectorSubcoreMesh(core_axis_name="c", subcore_axis_name="s")
    f = pl.kernel(
        _body,
        mesh=mesh,
        out_type=(
            jax.ShapeDtypeStruct((_B,), jnp.float32),
            jax.ShapeDtypeStruct((_B,), jnp.float32),
        ),
        scratch_types=[
            pltpu.VMEM((_BPW,), jnp.int32),
            pltpu.VMEM((_BPW,), jnp.int32),
            pltpu.VMEM((_BPW,), jnp.int32),
            pltpu.VMEM((_C, _K), jnp.float32),
            pltpu.VMEM((_C, _K), jnp.float32),
            pltpu.VMEM((_C, _K), jnp.float32),
            pltpu.VMEM((_BPW,), jnp.float32),
            pltpu.VMEM((_BPW,), jnp.float32),
            pltpu.SemaphoreType.DMA,
        ],
        compiler_params=pltpu.CompilerParams(
            needs_layout_passes=False, use_tc_tiling_on_sc=False
        ),
    )
    return f(users, items, neg_items, p, q)


def kernel(users, items, neg_items, P, Q):
    users = users.astype(jnp.int32)
    items = items.astype(jnp.int32)
    neg_items = neg_items.astype(jnp.int32)
    return _run(users, items, neg_items, P, Q)


# speedup vs baseline: 10.0346x; 1.7952x over previous
import jax
import jax.numpy as jnp
from jax import lax
from jax.experimental import pallas as pl
from jax.experimental.pallas import tpu as pltpu
from jax.experimental.pallas import tpu_sc as plsc

_M = 1000000
_K = 32
_B = 16384
_NC = 2
_NS = 16
_NW = _NC * _NS
_BPW = _B // _NW
_L = 16
_NST = _BPW // _L
_MP = 1000064


def _body(users_hbm, items_hbm, neg_hbm, pt_hbm, qt_hbm, pos_out, neg_out,
          idx_u, idx_i, idx_n, su, si, sn, buf, eu, ei, en, pos_v, neg_v,
          sem):
    wid = lax.axis_index("s") * _NC + lax.axis_index("c")
    base = wid * _BPW

    pltpu.sync_copy(users_hbm.at[pl.ds(base, _BPW)], idx_u)
    pltpu.sync_copy(items_hbm.at[pl.ds(base, _BPW)], idx_i)
    pltpu.sync_copy(neg_hbm.at[pl.ds(base, _BPW)], idx_n)

    def fetch_and_extract(tbl_hbm, smem_idx, vec_idx, dst, t):
        idxv = smem_idx[pl.ds(t * _L, _L)]
        blkv = (idxv >> 7) * 128
        copies = []
        for r in range(_L):
            off = pl.multiple_of(blkv[r], 128)
            copies.append(pltpu.async_copy(
                tbl_hbm.at[:, pl.ds(off, 128)],
                buf.at[pl.ds(r * _K, _K), :], sem))
        for cp in copies:
            cp.wait()
        lanes = jnp.bitwise_and(vec_idx[pl.ds(t * _L, _L)],
                                jnp.full((_L,), 127, jnp.int32))
        for k in range(_K):
            rows = jnp.arange(_L, dtype=jnp.int32) * _K + k
            dst[k, pl.ds(0, _L)] = plsc.load_gather(buf, [rows, lanes])

    def step(t, carry):
        fetch_and_extract(pt_hbm, idx_u, idx_u, eu, t)
        fetch_and_extract(qt_hbm, idx_i, idx_i, ei, t)
        fetch_and_extract(qt_hbm, idx_n, idx_n, en, t)
        acc_p = jnp.zeros((_L,), jnp.float32)
        acc_n = jnp.zeros((_L,), jnp.float32)
        for k in range(_K):
            uvec = eu[k, pl.ds(0, _L)]
            acc_p = acc_p + uvec * ei[k, pl.ds(0, _L)]
            acc_n = acc_n + uvec * en[k, pl.ds(0, _L)]
        o = pl.ds(t * _L, _L)
        pos_v[o] = acc_p
        neg_v[o] = acc_n
        return carry
    lax.fori_loop(0, _NST, step, 0)

    pltpu.sync_copy(pos_v, pos_out.at[pl.ds(base, _BPW)])
    pltpu.sync_copy(neg_v, neg_out.at[pl.ds(base, _BPW)])


@jax.jit
def _run(users, items, neg_items, p, q):
    mesh = plsc.VectorSubcoreMesh(core_axis_name="c", subcore_axis_name="s")
    f = pl.kernel(
        _body,
        mesh=mesh,
        out_type=(
            jax.ShapeDtypeStruct((_B,), jnp.float32),
            jax.ShapeDtypeStruct((_B,), jnp.float32),
        ),
        scratch_types=[
            pltpu.VMEM((_BPW,), jnp.int32),
            pltpu.VMEM((_BPW,), jnp.int32),
            pltpu.VMEM((_BPW,), jnp.int32),
            pltpu.SMEM((_BPW,), jnp.int32),
            pltpu.SMEM((_BPW,), jnp.int32),
            pltpu.SMEM((_BPW,), jnp.int32),
            pltpu.VMEM((_L * _K, 128), jnp.float32),
            pltpu.VMEM((_K, _L), jnp.float32),
            pltpu.VMEM((_K, _L), jnp.float32),
            pltpu.VMEM((_K, _L), jnp.float32),
            pltpu.VMEM((_BPW,), jnp.float32),
            pltpu.VMEM((_BPW,), jnp.float32),
            pltpu.SemaphoreType.DMA,
        ],
        compiler_params=pltpu.CompilerParams(
            needs_layout_passes=False, use_tc_tiling_on_sc=True
        ),
    )
    pt = jnp.pad(p.T, ((0, 0), (0, _MP - _M)))
    qt = jnp.pad(q.T, ((0, 0), (0, _MP - _M)))
    return f(users, items, neg_items, pt, qt)


def kernel(users, items, neg_items, P, Q):
    users = users.astype(jnp.int32)
    items = items.astype(jnp.int32)
    neg_items = neg_items.astype(jnp.int32)
    return _run(users, items, neg_items, P, Q)
